# baseline (speedup 1.0000x reference)
import functools

import jax
import jax.numpy as jnp
from jax import lax
from jax.experimental import pallas as pl
from jax.experimental.pallas import tpu as pltpu
from jax.experimental.pallas import tpu_sc as plsc

B = 16384
NF = 26
NCONT = 13
D = 32
FT = NF + NCONT
NW = 32
RW = B // NW
CH = 128
NCH = RW // CH
NG = RW // 16


def _body(xc_hbm, xnum_hbm, table_hbm, w_hbm, b_hbm, out_hbm,
          xblk_v, idxt_v, rows_v, xnum_v, w_v, b_v, num_v,
          gsem0, gsem1, wsem0, wsem1, wsem2, nsem0, nsem1):
    gsem = (gsem0, gsem1)
    wsem = (wsem0, wsem1, wsem2)
    nsem = (nsem0, nsem1)

    cid = lax.axis_index("c")
    sid = lax.axis_index("s")
    wid = sid * 2 + cid
    b0 = wid * RW

    pltpu.sync_copy(xc_hbm.at[pl.ds(b0, RW)], xblk_v)
    pltpu.sync_copy(xnum_hbm.at[pl.ds(b0, RW)], xnum_v)
    pltpu.sync_copy(w_hbm, w_v)
    pltpu.sync_copy(b_hbm, b_v)

    lanes = lax.iota(jnp.int32, 16)
    for f in range(NF):
        fcol = jnp.full((16,), f, jnp.int32)
        for g in range(NG):
            v = plsc.load_gather(xblk_v, [g * 16 + lanes, fcol])
            idxt_v[f, g // 8, pl.ds((g % 8) * 16, 16)] = v

    def fire_gathers(f):
        return [
            pltpu.async_copy(
                table_hbm.at[idxt_v.at[f, j]],
                rows_v.at[f % 3, pl.ds(j * CH, CH)],
                gsem[f % 2],
            )
            for j in range(NCH)
        ]

    def fire_write(f):
        return pltpu.async_copy(
            rows_v.at[f % 3], out_hbm.at[pl.ds(b0, RW), f], wsem[f % 3])

    gathers = fire_gathers(0)

    nwrites = {}
    for n in range(NCONT):
        w0 = w_v[n, pl.ds(0, 16)]
        w1 = w_v[n, pl.ds(16, 16)]
        a0 = b_v[n, pl.ds(0, 16)]
        a1 = b_v[n, pl.ds(16, 16)]
        ncol = jnp.full((16,), n, jnp.int32)
        if n >= 2:
            nwrites[n - 2].wait()

        def row(i, _, w0=w0, w1=w1, a0=a0, a1=a1, ncol=ncol, p=n % 2):
            s = plsc.load_gather(xnum_v, [jnp.full((16,), i, jnp.int32), ncol])
            num_v[p, i, pl.ds(0, 16)] = s * w0 + a0
            num_v[p, i, pl.ds(16, 16)] = s * w1 + a1
            return 0

        lax.fori_loop(0, RW, row, 0)
        nwrites[n] = pltpu.async_copy(
            num_v.at[n % 2], out_hbm.at[pl.ds(b0, RW), NF + n], nsem[n % 2])

    writes = {}
    for f in range(1, NF):
        if f >= 3:
            writes[f - 3].wait()
        prev = gathers
        gathers = fire_gathers(f)
        for c in prev:
            c.wait()
        writes[f - 1] = fire_write(f - 1)

    for c in gathers:
        c.wait()
    writes[NF - 1] = fire_write(NF - 1)
    for f in (NF - 3, NF - 2, NF - 1):
        writes[f].wait()
    nwrites[NCONT - 2].wait()
    nwrites[NCONT - 1].wait()


_embed = functools.partial(
    pl.kernel,
    out_type=jax.ShapeDtypeStruct((B, FT, D), jnp.float32),
    mesh=plsc.VectorSubcoreMesh(core_axis_name="c", subcore_axis_name="s"),
    compiler_params=pltpu.CompilerParams(
        use_tc_tiling_on_sc=False, needs_layout_passes=False
    ),
    scratch_types=[
        pltpu.VMEM((RW, NF), jnp.int32),
        pltpu.VMEM((NF, NCH, CH), jnp.int32),
        pltpu.VMEM((3, RW, D), jnp.float32),
        pltpu.VMEM((RW, NCONT), jnp.float32),
        pltpu.VMEM((NCONT, D), jnp.float32),
        pltpu.VMEM((NCONT, D), jnp.float32),
        pltpu.VMEM((2, RW, D), jnp.float32),
    ] + [pltpu.SemaphoreType.DMA] * 7,
)(_body)


def kernel(x_categ, x_numer, embed_table, num_weights, num_biases):
    return _embed(x_categ.astype(jnp.int32), x_numer, embed_table,
                  num_weights, num_biases)
